# baseline (speedup 1.0000x reference)
import functools
import jax
import jax.numpy as jnp
from jax import lax
from jax.experimental import pallas as pl
from jax.experimental.pallas import tpu as pltpu
from jax.experimental.pallas import tpu_sc as plsc


def _prep_body(wt_ref, base_ref):
    C, K, L = base_ref.shape
    E = wt_ref.shape[0]
    PE_C = C - E
    c_i = lax.broadcasted_iota(jnp.int32, (PE_C, L), 0)
    l_f = lax.broadcasted_iota(jnp.int32, (PE_C, L), 1).astype(jnp.float32)
    pair = (c_i // 2).astype(jnp.float32)
    div = jnp.exp(pair * (-jnp.log(10000.0) * 2.0 / PE_C))
    angle = l_f * div
    pe = jnp.where((c_i % 2) == 0, jnp.sin(angle), jnp.cos(angle))
    base_ref[0:PE_C, :, :] = jnp.broadcast_to(pe[:, None, :], (PE_C, K, L))
    base_ref[PE_C:C, :, :] = jnp.broadcast_to(wt_ref[...], (E, K, L))


def kernel(cond_mask, embed_weight):
    B, _, K, L = cond_mask.shape
    E = embed_weight.shape[1]
    C = 128 + E
    PE_C = 128
    wt = embed_weight.T[:, :, None]

    base = pl.pallas_call(
        _prep_body,
        out_shape=jax.ShapeDtypeStruct((C, K, L), jnp.float32),
    )(wt)

    info = plsc.get_sparse_core_info()
    NC, NS = info.num_cores, info.num_subcores
    NW = NC * NS
    tc_per_w = PE_C // NW

    mesh = plsc.VectorSubcoreMesh(core_axis_name="c", subcore_axis_name="s")

    @functools.partial(
        pl.kernel,
        mesh=mesh,
        out_type=jax.ShapeDtypeStruct((B, C, K, L), jnp.float32),
        scratch_types=[
            pltpu.VMEM((K, L), jnp.float32),
            pltpu.SemaphoreType.DMA,
        ],
    )
    def sc_fanout(base_hbm, out_hbm, plane, sem):
        wid = lax.axis_index("s") * NC + lax.axis_index("c")

        def fan_out(c):
            pltpu.sync_copy(base_hbm.at[c], plane)
            copies = [
                pltpu.async_copy(plane, out_hbm.at[b, c], sem)
                for b in range(B)
            ]
            for cp in copies:
                cp.wait()

        for i in range(tc_per_w):
            fan_out(wid * tc_per_w + i)

        @pl.when(wid < E)
        def _():
            fan_out(PE_C + wid)

    return sc_fanout(base)


# speedup vs baseline: 1.0646x; 1.0646x over previous
import functools
import jax
import jax.numpy as jnp
from jax import lax
from jax.experimental import pallas as pl
from jax.experimental.pallas import tpu as pltpu
from jax.experimental.pallas import tpu_sc as plsc


def _prep_body(wt_ref, base_ref):
    C, K, L = base_ref.shape
    E = wt_ref.shape[0]
    PE_C = C - E
    c_i = lax.broadcasted_iota(jnp.int32, (PE_C, L), 0)
    l_f = lax.broadcasted_iota(jnp.int32, (PE_C, L), 1).astype(jnp.float32)
    pair = (c_i // 2).astype(jnp.float32)
    div = jnp.exp(pair * (-jnp.log(10000.0) * 2.0 / PE_C))
    angle = l_f * div
    pe = jnp.where((c_i % 2) == 0, jnp.sin(angle), jnp.cos(angle))
    base_ref[0:PE_C, :, :] = jnp.broadcast_to(pe[:, None, :], (PE_C, K, L))
    base_ref[PE_C:C, :, :] = jnp.broadcast_to(wt_ref[...], (E, K, L))


def kernel(cond_mask, embed_weight):
    B, _, K, L = cond_mask.shape
    E = embed_weight.shape[1]
    C = 128 + E
    PE_C = 128
    wt = embed_weight.T[:, :, None]

    base = pl.pallas_call(
        _prep_body,
        out_shape=jax.ShapeDtypeStruct((C, K, L), jnp.float32),
    )(wt)

    info = plsc.get_sparse_core_info()
    NC, NS = info.num_cores, info.num_subcores
    NW = NC * NS
    tc_per_w = PE_C // NW
    half_b = B // 2

    mesh = plsc.VectorSubcoreMesh(core_axis_name="c", subcore_axis_name="s")

    @functools.partial(
        pl.kernel,
        mesh=mesh,
        out_type=jax.ShapeDtypeStruct((B, C, K, L), jnp.float32),
        scratch_types=[
            pltpu.VMEM((K, L), jnp.float32),
            pltpu.VMEM((K, L), jnp.float32),
            pltpu.SemaphoreType.DMA,
            pltpu.SemaphoreType.DMA,
            pltpu.SemaphoreType.DMA,
            pltpu.SemaphoreType.DMA,
        ],
    )
    def sc_fanout(base_hbm, out_hbm, buf0, buf1, ss0, ss1, ws0, ws1):
        wid = lax.axis_index("s") * NC + lax.axis_index("c")
        bufs = [buf0, buf1]
        ssems = [ss0, ss1]
        wsems = [ws0, ws1]

        fb0 = (wid % 2) * half_b
        units = [(wid * tc_per_w + i, list(range(B))) for i in range(tc_per_w)]
        units.append((PE_C + wid // 2, [fb0 + j for j in range(half_b)]))

        stage = [None, None]
        whandles = [[], []]
        stage[0] = pltpu.async_copy(base_hbm.at[units[0][0]], bufs[0], ssems[0])
        for i, (c, bs) in enumerate(units):
            p = i % 2
            if i + 1 < len(units):
                for h in whandles[1 - p]:
                    h.wait()
                whandles[1 - p] = []
                stage[1 - p] = pltpu.async_copy(
                    base_hbm.at[units[i + 1][0]], bufs[1 - p], ssems[1 - p]
                )
            stage[p].wait()
            whandles[p] = [
                pltpu.async_copy(bufs[p], out_hbm.at[b, c], wsems[p])
                for b in bs
            ]
        for p in (0, 1):
            for h in whandles[p]:
                h.wait()

    return sc_fanout(base)


# speedup vs baseline: 1.0793x; 1.0138x over previous
import functools
import jax
import jax.numpy as jnp
from jax import lax
from jax.experimental import pallas as pl
from jax.experimental.pallas import tpu as pltpu
from jax.experimental.pallas import tpu_sc as plsc


def _prep_body(wt_ref, base_ref):
    C, Kb, L = base_ref.shape
    E = wt_ref.shape[0]
    PE_C = C - E
    c_i = lax.broadcasted_iota(jnp.int32, (PE_C, L), 0)
    l_f = lax.broadcasted_iota(jnp.int32, (PE_C, L), 1).astype(jnp.float32)
    pair = (c_i // 2).astype(jnp.float32)
    div = jnp.exp(pair * (-jnp.log(10000.0) * 2.0 / PE_C))
    angle = l_f * div
    pe = jnp.where((c_i % 2) == 0, jnp.sin(angle), jnp.cos(angle))
    base_ref[0:PE_C, :, :] = jnp.broadcast_to(pe[:, None, :], (PE_C, Kb, L))
    base_ref[PE_C:C, :, :] = jnp.broadcast_to(wt_ref[...], (E, Kb, L))


def kernel(cond_mask, embed_weight):
    B, _, K, L = cond_mask.shape
    E = embed_weight.shape[1]
    C = 128 + E
    PE_C = 128
    wt = embed_weight.T[:, :, None]

    KB = 4
    base = pl.pallas_call(
        _prep_body,
        grid=(KB,),
        in_specs=[pl.BlockSpec((E, K // KB, 1), lambda kb: (0, kb, 0))],
        out_specs=pl.BlockSpec((C, K // KB, L), lambda kb: (0, kb, 0)),
        out_shape=jax.ShapeDtypeStruct((C, K, L), jnp.float32),
    )(wt)

    info = plsc.get_sparse_core_info()
    NC, NS = info.num_cores, info.num_subcores
    NW = NC * NS
    tc_per_w = PE_C // NW
    half_b = B // 2

    mesh = plsc.VectorSubcoreMesh(core_axis_name="c", subcore_axis_name="s")

    @functools.partial(
        pl.kernel,
        mesh=mesh,
        out_type=jax.ShapeDtypeStruct((B, C, K, L), jnp.float32),
        scratch_types=[
            pltpu.VMEM((K, L), jnp.float32),
            pltpu.VMEM((K, L), jnp.float32),
            pltpu.SemaphoreType.DMA,
            pltpu.SemaphoreType.DMA,
            pltpu.SemaphoreType.DMA,
            pltpu.SemaphoreType.DMA,
        ],
    )
    def sc_fanout(base_hbm, out_hbm, buf0, buf1, ss0, ss1, ws0, ws1):
        wid = lax.axis_index("s") * NC + lax.axis_index("c")
        bufs = [buf0, buf1]
        ssems = [ss0, ss1]
        wsems = [ws0, ws1]

        fb0 = (wid % 2) * half_b
        units = [(wid * tc_per_w + i, list(range(B))) for i in range(tc_per_w)]
        units.append((PE_C + wid // 2, [fb0 + j for j in range(half_b)]))

        stage = [None, None]
        whandles = [[], []]
        stage[0] = pltpu.async_copy(base_hbm.at[units[0][0]], bufs[0], ssems[0])
        for i, (c, bs) in enumerate(units):
            p = i % 2
            if i + 1 < len(units):
                for h in whandles[1 - p]:
                    h.wait()
                whandles[1 - p] = []
                stage[1 - p] = pltpu.async_copy(
                    base_hbm.at[units[i + 1][0]], bufs[1 - p], ssems[1 - p]
                )
            stage[p].wait()
            whandles[p] = [
                pltpu.async_copy(bufs[p], out_hbm.at[b, c], wsems[p])
                for b in bs
            ]
        for p in (0, 1):
            for h in whandles[p]:
                h.wait()

    return sc_fanout(base)


# speedup vs baseline: 1.2313x; 1.1408x over previous
import functools
import jax
import jax.numpy as jnp
from jax import lax
from jax.experimental import pallas as pl
from jax.experimental.pallas import tpu as pltpu
from jax.experimental.pallas import tpu_sc as plsc

_NLANES = 16


def _prep_body(wt_ref, pe_ref, wseed_ref):
    PE_C, L = pe_ref.shape
    E, K, _ = wseed_ref.shape
    c_i = lax.broadcasted_iota(jnp.int32, (PE_C, L), 0)
    l_f = lax.broadcasted_iota(jnp.int32, (PE_C, L), 1).astype(jnp.float32)
    pair = (c_i // 2).astype(jnp.float32)
    div = jnp.exp(pair * (-jnp.log(10000.0) * 2.0 / PE_C))
    angle = l_f * div
    pe_ref[...] = jnp.where((c_i % 2) == 0, jnp.sin(angle), jnp.cos(angle))
    wseed_ref[...] = jnp.broadcast_to(wt_ref[...], (E, K, _NLANES))


def kernel(cond_mask, embed_weight):
    B, _, K, L = cond_mask.shape
    E = embed_weight.shape[1]
    C = 128 + E
    PE_C = 128
    wt = embed_weight.T[:, :, None]

    pe, wseed = pl.pallas_call(
        _prep_body,
        out_shape=[
            jax.ShapeDtypeStruct((PE_C, L), jnp.float32),
            jax.ShapeDtypeStruct((E, K, _NLANES), jnp.float32),
        ],
    )(wt)

    info = plsc.get_sparse_core_info()
    NC, NS = info.num_cores, info.num_subcores
    NW = NC * NS
    tc_per_w = PE_C // NW
    half_b = B // 2
    n_grp = L // _NLANES
    ROWS_PER_STEP = 4

    mesh = plsc.VectorSubcoreMesh(core_axis_name="c", subcore_axis_name="s")

    @functools.partial(
        pl.kernel,
        mesh=mesh,
        out_type=jax.ShapeDtypeStruct((B, C, K, L), jnp.float32),
        scratch_types=[
            pltpu.VMEM((K, L), jnp.float32),
            pltpu.VMEM((K, L), jnp.float32),
            pltpu.VMEM((L,), jnp.float32),
            pltpu.VMEM((K, _NLANES), jnp.float32),
            pltpu.SemaphoreType.DMA,
            pltpu.SemaphoreType.DMA,
        ],
    )
    def sc_fanout(pe_hbm, wseed_hbm, out_hbm, buf0, buf1, rowbuf, seedbuf,
                  ws0, ws1):
        wid = lax.axis_index("s") * NC + lax.axis_index("c")
        bufs = [buf0, buf1]
        wsems = [ws0, ws1]

        def fill_time(buf, c):
            pltpu.sync_copy(pe_hbm.at[c], rowbuf)
            vs = [rowbuf[pl.ds(_NLANES * j, _NLANES)] for j in range(n_grp)]

            def row_step(t, _):
                for r in range(ROWS_PER_STEP):
                    k = t * ROWS_PER_STEP + r
                    for j in range(n_grp):
                        buf[k, pl.ds(_NLANES * j, _NLANES)] = vs[j]
                return _

            lax.fori_loop(0, K // ROWS_PER_STEP, row_step, None)

        def fill_feat(buf, e):
            pltpu.sync_copy(wseed_hbm.at[e], seedbuf)

            def row_step(t, _):
                for r in range(ROWS_PER_STEP):
                    k = t * ROWS_PER_STEP + r
                    v = seedbuf[k, :]
                    for j in range(n_grp):
                        buf[k, pl.ds(_NLANES * j, _NLANES)] = v
                return _

            lax.fori_loop(0, K // ROWS_PER_STEP, row_step, None)

        fb0 = (wid % 2) * half_b
        units = [(True, wid * tc_per_w + i, list(range(B)))
                 for i in range(tc_per_w)]
        units.append((False, wid // 2, [fb0 + j for j in range(half_b)]))

        whandles = [[], []]
        for i, (is_time, idx, bs) in enumerate(units):
            p = i % 2
            for h in whandles[p]:
                h.wait()
            whandles[p] = []
            if is_time:
                fill_time(bufs[p], idx)
                c = idx
            else:
                fill_feat(bufs[p], idx)
                c = PE_C + idx
            whandles[p] = [
                pltpu.async_copy(bufs[p], out_hbm.at[b, c], wsems[p])
                for b in bs
            ]
        for p in (0, 1):
            for h in whandles[p]:
                h.wait()

    return sc_fanout(pe, wseed)


# speedup vs baseline: 1.2563x; 1.0203x over previous
import functools
import jax
import jax.numpy as jnp
from jax import lax
from jax.experimental import pallas as pl
from jax.experimental.pallas import tpu as pltpu
from jax.experimental.pallas import tpu_sc as plsc

_NLANES = 16


def _prep_body(wt_ref, pe_ref, wseed_ref):
    PE_C, L = pe_ref.shape
    E, K, _ = wseed_ref.shape
    c_i = lax.broadcasted_iota(jnp.int32, (PE_C, L), 0)
    l_f = lax.broadcasted_iota(jnp.int32, (PE_C, L), 1).astype(jnp.float32)
    pair = (c_i // 2).astype(jnp.float32)
    div = jnp.exp(pair * (-jnp.log(10000.0) * 2.0 / PE_C))
    angle = l_f * div
    pe_ref[...] = jnp.where((c_i % 2) == 0, jnp.sin(angle), jnp.cos(angle))
    wseed_ref[...] = jnp.broadcast_to(wt_ref[...], (E, K, _NLANES))


def kernel(cond_mask, embed_weight):
    B, _, K, L = cond_mask.shape
    E = embed_weight.shape[1]
    C = 128 + E
    PE_C = 128
    wt = embed_weight.T[:, :, None]

    pe, wseed = pl.pallas_call(
        _prep_body,
        out_shape=[
            jax.ShapeDtypeStruct((PE_C, L), jnp.float32),
            jax.ShapeDtypeStruct((E, K, _NLANES), jnp.float32),
        ],
    )(wt)

    info = plsc.get_sparse_core_info()
    NC, NS = info.num_cores, info.num_subcores
    NW = NC * NS
    tc_per_w = PE_C // NW
    half_b = B // 2
    n_grp = L // _NLANES
    ROWS_PER_STEP = 4

    mesh = plsc.VectorSubcoreMesh(core_axis_name="c", subcore_axis_name="s")

    @functools.partial(
        pl.kernel,
        mesh=mesh,
        out_type=jax.ShapeDtypeStruct((B, C, K, L), jnp.float32),
        scratch_types=[
            pltpu.VMEM((K, L), jnp.float32),
            pltpu.VMEM((K, L), jnp.float32),
            pltpu.VMEM((K, L), jnp.float32),
            pltpu.VMEM((L,), jnp.float32),
            pltpu.VMEM((K, _NLANES), jnp.float32),
            pltpu.SemaphoreType.DMA,
            pltpu.SemaphoreType.DMA,
            pltpu.SemaphoreType.DMA,
        ],
    )
    def sc_fanout(pe_hbm, wseed_hbm, out_hbm, buf0, buf1, buf2, rowbuf,
                  seedbuf, ws0, ws1, ws2):
        wid = lax.axis_index("s") * NC + lax.axis_index("c")
        bufs = [buf0, buf1, buf2]
        wsems = [ws0, ws1, ws2]

        def fill_time(buf, c):
            pltpu.sync_copy(pe_hbm.at[c], rowbuf)
            vs = [rowbuf[pl.ds(_NLANES * j, _NLANES)] for j in range(n_grp)]

            def row_step(t, _):
                for r in range(ROWS_PER_STEP):
                    k = t * ROWS_PER_STEP + r
                    for j in range(n_grp):
                        buf[k, pl.ds(_NLANES * j, _NLANES)] = vs[j]
                return _

            lax.fori_loop(0, K // ROWS_PER_STEP, row_step, None)

        def fill_feat(buf, e):
            pltpu.sync_copy(wseed_hbm.at[e], seedbuf)

            def row_step(t, _):
                for r in range(ROWS_PER_STEP):
                    k = t * ROWS_PER_STEP + r
                    v = seedbuf[k, :]
                    for j in range(n_grp):
                        buf[k, pl.ds(_NLANES * j, _NLANES)] = v
                return _

            lax.fori_loop(0, K // ROWS_PER_STEP, row_step, None)

        fb0 = (wid % 2) * half_b
        units = [(True, wid * tc_per_w + i, list(range(B)))
                 for i in range(tc_per_w)]
        units.append((False, wid // 2, [fb0 + j for j in range(half_b)]))

        whandles = [[], [], []]
        for i, (is_time, idx, bs) in enumerate(units):
            p = i % 3
            for h in whandles[p]:
                h.wait()
            whandles[p] = []
            if is_time:
                fill_time(bufs[p], idx)
                c = idx
            else:
                fill_feat(bufs[p], idx)
                c = PE_C + idx
            whandles[p] = [
                pltpu.async_copy(bufs[p], out_hbm.at[b, c], wsems[p])
                for b in bs
            ]
        for p in (0, 1, 2):
            for h in whandles[p]:
                h.wait()

    return sc_fanout(pe, wseed)
